# Initial kernel scaffold; baseline (speedup 1.0000x reference)
import jax
import jax.numpy as jnp
from jax.experimental import pallas as pl


def kernel(x, edge_index, batch, W0, b0, bn0_g, bn0_b, W1, b1, bn1_g, bn1_b, W2, b2):
    raise NotImplementedError("write your pallas kernel here")


# speedup vs baseline: 18.6610x; 18.6610x over previous
import functools

import jax
import jax.numpy as jnp
from jax import lax
from jax.experimental import pallas as pl
from jax.experimental.pallas import tpu as pltpu
from jax.experimental.pallas import tpu_sc as plsc

N = 10000
E = 320000
D_IN = 128
D_H = 512
D_C1 = 1024
NCLS = 128
NG = 64

NCORES = 2
NSUB = 16
NW = NCORES * NSUB

CHUNK = 128
NCH = -(-E // (NW * CHUNK))
EPW = NCH * CHUNK
EPAD = NW * EPW

NPAD = 10240
RPT = NPAD // NSUB
DEGW = 16

def _deg_body(dst_hbm, zeros_hbm, ones_hbm, out_hbm, idx_v, ones_v, deg_sh):
    c = lax.axis_index("c")
    s = lax.axis_index("s")
    wid = s * NCORES + c
    pltpu.sync_copy(ones_hbm, ones_v)
    pltpu.sync_copy(zeros_hbm, deg_sh.at[pl.ds(s * RPT, RPT)])
    plsc.subcore_barrier()

    def step(i, carry):
        off = wid * EPW + i * CHUNK
        pltpu.sync_copy(dst_hbm.at[pl.ds(off, CHUNK)], idx_v)
        pltpu.sync_copy(ones_v, deg_sh.at[idx_v], add=True)
        return carry

    lax.fori_loop(0, NCH, step, 0)
    plsc.subcore_barrier()
    pltpu.sync_copy(deg_sh.at[pl.ds(s * RPT, RPT)],
                    out_hbm.at[c, pl.ds(s * RPT, RPT)])


@functools.cache
def _sc_calls():
    mesh = plsc.VectorSubcoreMesh(core_axis_name="c", subcore_axis_name="s",
                                  num_cores=NCORES, num_subcores=NSUB)
    deg_call = pl.kernel(
        _deg_body,
        out_type=jax.ShapeDtypeStruct((NCORES, NPAD, DEGW), jnp.float32),
        mesh=mesh,
        scratch_types=[
            pltpu.VMEM((CHUNK,), jnp.int32),
            pltpu.VMEM((CHUNK, DEGW), jnp.float32),
            pltpu.VMEM_SHARED((NPAD, DEGW), jnp.float32),
        ],
    )
    agg_call = pl.kernel(
        _agg_body,
        out_type=jax.ShapeDtypeStruct((NCORES, NPAD, D_IN), jnp.float32),
        mesh=mesh,
        scratch_types=[
            pltpu.VMEM((CHUNK,), jnp.int32),
            pltpu.VMEM((CHUNK,), jnp.int32),
            pltpu.VMEM((CHUNK, D_IN), jnp.float32),
            pltpu.VMEM_SHARED((NPAD, D_IN), jnp.float32),
            pltpu.SemaphoreType.DMA,
        ],
    )
    return deg_call, agg_call


def _agg_body(src_hbm, dst_hbm, y_hbm, zeros_hbm, out_hbm,
              isrc, idst, rows, s_sh, sem):
    c = lax.axis_index("c")
    s = lax.axis_index("s")
    wid = s * NCORES + c
    pltpu.sync_copy(zeros_hbm, s_sh.at[pl.ds(s * RPT, RPT)])
    plsc.subcore_barrier()

    def step(i, carry):
        off = wid * EPW + i * CHUNK
        pltpu.sync_copy(src_hbm.at[pl.ds(off, CHUNK)], isrc)
        pltpu.sync_copy(dst_hbm.at[pl.ds(off, CHUNK)], idst)
        pltpu.async_copy(y_hbm.at[isrc], rows, sem).wait()
        pltpu.sync_copy(rows, s_sh.at[idst], add=True)
        return carry

    lax.fori_loop(0, NCH, step, 0)
    plsc.subcore_barrier()
    pltpu.sync_copy(s_sh.at[pl.ds(s * RPT, RPT)],
                    out_hbm.at[c, pl.ds(s * RPT, RPT)])


RB = 1024


def _y_body(deg_ref, x_ref, y_ref):
    d = deg_ref[...]
    cnt = d[0, :, 0:1] + d[1, :, 0:1] + 1.0
    y_ref[...] = x_ref[...] * lax.rsqrt(cnt)


def _y_call(deg, x_p):
    return pl.pallas_call(
        _y_body,
        grid=(NPAD // RB,),
        in_specs=[
            pl.BlockSpec((NCORES, RB, DEGW), lambda i: (0, i, 0)),
            pl.BlockSpec((RB, D_IN), lambda i: (i, 0)),
        ],
        out_specs=pl.BlockSpec((RB, D_IN), lambda i: (i, 0)),
        out_shape=jax.ShapeDtypeStruct((NPAD, D_IN), jnp.float32),
    )(deg, x_p)


RD = 1000
NBLK = N // RD


def _dense_body(parts_ref, y_ref, deg_ref, batch_ref, w0_ref, sh0_ref,
                w1_ref, sh1_ref, w2_ref, b2_ref, out_ref,
                pooled_acc, cnt_acc):
    i = pl.program_id(0)

    @pl.when(i == 0)
    def _init():
        pooled_acc[...] = jnp.zeros_like(pooled_acc)
        cnt_acc[...] = jnp.zeros_like(cnt_acc)

    d = deg_ref[...]
    cnt = d[0, :, 0:1] + d[1, :, 0:1] + 1.0
    dinv = lax.rsqrt(cnt)
    p = parts_ref[...]
    s = p[0] + p[1] + y_ref[...]
    agg = s * dinv
    h = jnp.dot(agg, w0_ref[...], preferred_element_type=jnp.float32)
    h = h * sh0_ref[0:1, :] + sh0_ref[1:2, :]
    h = jnp.maximum(h, 0.0)

    bids = batch_ref[...].reshape(1, RD)
    gid = lax.broadcasted_iota(jnp.int32, (NG, RD), 0)
    oh = (gid == bids).astype(jnp.float32)
    pooled_acc[...] += lax.dot_general(
        oh, h, (((1,), (0,)), ((), ())),
        preferred_element_type=jnp.float32)
    cnt_acc[...] += jnp.sum(oh, axis=1, keepdims=True)

    @pl.when(i == NBLK - 1)
    def _finish():
        pooled = pooled_acc[...] / jnp.maximum(cnt_acc[...], 1.0)
        z = jnp.dot(pooled, w1_ref[...], preferred_element_type=jnp.float32)
        z = z * sh1_ref[0:1, :] + sh1_ref[1:2, :]
        z = jnp.maximum(z, 0.0)
        z = jnp.dot(z, w2_ref[...], preferred_element_type=jnp.float32)
        out_ref[...] = z + b2_ref[...]


def _dense_call(parts, y_p, deg, batch3, W0, sh0, W1, sh1, W2, b2r):
    return pl.pallas_call(
        _dense_body,
        grid=(NBLK,),
        in_specs=[
            pl.BlockSpec((NCORES, RD, D_IN), lambda i: (0, i, 0)),
            pl.BlockSpec((RD, D_IN), lambda i: (i, 0)),
            pl.BlockSpec((NCORES, RD, DEGW), lambda i: (0, i, 0)),
            pl.BlockSpec((1, 1, RD), lambda i: (i, 0, 0)),
            pl.BlockSpec((D_IN, D_H), lambda i: (0, 0)),
            pl.BlockSpec((2, D_H), lambda i: (0, 0)),
            pl.BlockSpec((D_H, D_C1), lambda i: (0, 0)),
            pl.BlockSpec((2, D_C1), lambda i: (0, 0)),
            pl.BlockSpec((D_C1, NCLS), lambda i: (0, 0)),
            pl.BlockSpec((1, NCLS), lambda i: (0, 0)),
        ],
        out_specs=pl.BlockSpec((NG, NCLS), lambda i: (0, 0)),
        out_shape=jax.ShapeDtypeStruct((NG, NCLS), jnp.float32),
        scratch_shapes=[
            pltpu.VMEM((NG, D_H), jnp.float32),
            pltpu.VMEM((NG, 1), jnp.float32),
        ],
    )(parts, y_p, deg, batch3, W0, sh0, W1, sh1, W2, b2r)


def kernel(x, edge_index, batch, W0, b0, bn0_g, bn0_b, W1, b1, bn1_g, bn1_b,
           W2, b2):
    f32 = jnp.float32
    src = edge_index[0]
    dst = edge_index[1]
    pad = jnp.full((EPAD - E,), N, dtype=jnp.int32)
    src_p = jnp.concatenate([src, pad])
    dst_p = jnp.concatenate([dst, pad])
    x_p = jnp.zeros((NPAD, D_IN), f32).at[:N].set(x)
    batch3 = batch.reshape(NBLK, 1, RD)

    ones = jnp.ones((CHUNK, DEGW), f32)
    zeros_deg = jnp.zeros((RPT, DEGW), f32)
    zeros_row = jnp.zeros((RPT, D_IN), f32)

    deg_call, agg_call = _sc_calls()
    deg = deg_call(dst_p, zeros_deg, ones)
    y_p = _y_call(deg, x_p)
    parts = agg_call(src_p, dst_p, y_p, zeros_row)

    c = lax.rsqrt(jnp.asarray(1.0 + 1e-5, f32))
    sh0 = jnp.stack([c * bn0_g, b0 * c * bn0_g + bn0_b])
    sh1 = jnp.stack([c * bn1_g, b1 * c * bn1_g + bn1_b])
    b2r = b2.reshape(1, NCLS)

    return _dense_call(parts, y_p, deg, batch3, W0, sh0, W1, sh1, W2, b2r)


# speedup vs baseline: 23.0327x; 1.2343x over previous
import functools

import jax
import jax.numpy as jnp
from jax import lax
from jax.experimental import pallas as pl
from jax.experimental.pallas import tpu as pltpu
from jax.experimental.pallas import tpu_sc as plsc

N = 10000
E = 320000
D_IN = 128
D_H = 512
D_C1 = 1024
NCLS = 128
NG = 64

NCORES = 2
NSUB = 16
NW = NCORES * NSUB

CHUNK = 64
NCH = 160
EPW = NCH * CHUNK
EPAD = NW * EPW
NOUT = NCH // 8

NPAD = 10240
RPT = NPAD // NSUB

def _deg_body(dst_hbm, zeros_hbm, ones_hbm, out_hbm, ij, ones_v, deg_sh,
              sem_i, sem_s):
    c = lax.axis_index("c")
    s = lax.axis_index("s")
    wid = s * NCORES + c
    base = wid * EPW
    pltpu.sync_copy(ones_hbm, ones_v)
    pltpu.sync_copy(zeros_hbm, deg_sh.at[pl.ds(s * RPT, RPT)])

    def load(j, slot):
        pltpu.async_copy(dst_hbm.at[pl.ds(base + j * CHUNK, CHUNK)],
                         ij[slot], sem_i[slot])

    def wait_idx(slot):
        pltpu.make_async_copy(dst_hbm.at[pl.ds(base, CHUNK)],
                              ij[slot], sem_i[slot]).wait()

    for k in range(4):
        load(k, k)
    plsc.subcore_barrier()

    def outer(g, carry):
        for b in range(8):
            s8, s4 = b % 8, b % 4
            wait_idx(s8)

            def w_s():
                pltpu.make_async_copy(
                    ones_v, deg_sh.at[ij[(b + 4) % 8]],
                    sem_s[s4]).wait()

            if b >= 4:
                w_s()
            else:
                pl.when(g >= 1)(w_s)
            pltpu.async_copy(ones_v, deg_sh.at[ij[s8]], sem_s[s4],
                             add=True)

            def l_n():
                load(g * 8 + b + 4, (b + 4) % 8)

            if b < 4:
                l_n()
            else:
                pl.when(g < NOUT - 1)(l_n)
        return carry

    lax.fori_loop(0, NOUT, outer, 0)
    for k in range(4):
        pltpu.make_async_copy(ones_v, deg_sh.at[ij[(k + 4) % 8]],
                              sem_s[k]).wait()
    plsc.subcore_barrier()
    pltpu.sync_copy(deg_sh.at[pl.ds(s * RPT, RPT)],
                    out_hbm.at[c, pl.ds(s * RPT, RPT)])


@functools.cache
def _sc_calls():
    mesh = plsc.VectorSubcoreMesh(core_axis_name="c", subcore_axis_name="s",
                                  num_cores=NCORES, num_subcores=NSUB)
    deg_call = pl.kernel(
        _deg_body,
        out_type=jax.ShapeDtypeStruct((NCORES, NPAD), jnp.float32),
        mesh=mesh,
        scratch_types=[
            [pltpu.VMEM((CHUNK,), jnp.int32) for _ in range(8)],
            pltpu.VMEM((CHUNK,), jnp.float32),
            pltpu.VMEM_SHARED((NPAD,), jnp.float32),
            [pltpu.SemaphoreType.DMA for _ in range(8)],
            [pltpu.SemaphoreType.DMA for _ in range(4)],
        ],
    )
    agg_call = pl.kernel(
        _agg_body,
        out_type=jax.ShapeDtypeStruct((NCORES, NPAD, D_IN), jnp.float32),
        mesh=mesh,
        scratch_types=[
            [pltpu.VMEM((CHUNK,), jnp.int32) for _ in range(8)],
            [pltpu.VMEM((CHUNK,), jnp.int32) for _ in range(8)],
            [pltpu.VMEM((CHUNK, D_IN), jnp.float32) for _ in range(4)],
            pltpu.VMEM_SHARED((NPAD, D_IN), jnp.float32),
            [pltpu.SemaphoreType.DMA for _ in range(8)],
            [pltpu.SemaphoreType.DMA for _ in range(4)],
            [pltpu.SemaphoreType.DMA for _ in range(4)],
        ],
    )
    return deg_call, agg_call


def _agg_body(src_hbm, dst_hbm, y_hbm, zeros_hbm, out_hbm, isrc, idst, rows,
              s_sh, sem_i, sem_g, sem_s):
    c = lax.axis_index("c")
    s = lax.axis_index("s")
    wid = s * NCORES + c
    base = wid * EPW
    pltpu.sync_copy(zeros_hbm, s_sh.at[pl.ds(s * RPT, RPT)])

    def load(j, slot):
        off = base + j * CHUNK
        pltpu.async_copy(src_hbm.at[pl.ds(off, CHUNK)], isrc[slot],
                         sem_i[slot])
        pltpu.async_copy(dst_hbm.at[pl.ds(off, CHUNK)], idst[slot],
                         sem_i[slot])

    def wait_idx(slot):
        pltpu.make_async_copy(src_hbm.at[pl.ds(base, CHUNK)], isrc[slot],
                              sem_i[slot]).wait()
        pltpu.make_async_copy(dst_hbm.at[pl.ds(base, CHUNK)], idst[slot],
                              sem_i[slot]).wait()

    def gather(slot8, slot4):
        pltpu.async_copy(y_hbm.at[isrc[slot8]], rows[slot4], sem_g[slot4])

    for k in range(6):
        load(k, k)
    wait_idx(0)
    gather(0, 0)
    wait_idx(1)
    gather(1, 1)
    plsc.subcore_barrier()

    def outer(g, carry):
        for b in range(8):
            s8, s4 = b % 8, b % 4
            pltpu.make_async_copy(y_hbm.at[isrc[s8]], rows[s4],
                                  sem_g[s4]).wait()
            pltpu.async_copy(rows[s4], s_sh.at[idst[s8]], sem_s[s4],
                             add=True)

            def w_s():
                pltpu.make_async_copy(
                    rows[(b + 2) % 4], s_sh.at[idst[(b + 6) % 8]],
                    sem_s[(b + 2) % 4]).wait()

            if b >= 2:
                w_s()
            else:
                pl.when(g >= 1)(w_s)

            def g_n():
                wait_idx((b + 2) % 8)
                gather((b + 2) % 8, (b + 2) % 4)

            if b < 6:
                g_n()
            else:
                pl.when(g < NOUT - 1)(g_n)

            def l_n():
                load(g * 8 + b + 6, (b + 6) % 8)

            if b < 2:
                l_n()
            else:
                pl.when(g < NOUT - 1)(l_n)
        return carry

    lax.fori_loop(0, NOUT, outer, 0)
    pltpu.make_async_copy(rows[2], s_sh.at[idst[6]], sem_s[2]).wait()
    pltpu.make_async_copy(rows[3], s_sh.at[idst[7]], sem_s[3]).wait()
    plsc.subcore_barrier()
    pltpu.sync_copy(s_sh.at[pl.ds(s * RPT, RPT)],
                    out_hbm.at[c, pl.ds(s * RPT, RPT)])


RB = 1024


def _y_body(deg_ref, x_ref, y_ref, dinv_ref):
    d = deg_ref[...]
    cnt = d[0:1, :] + d[1:2, :] + 1.0
    dinv = jnp.transpose(lax.rsqrt(cnt))
    y_ref[...] = x_ref[...] * dinv
    dinv_ref[...] = jnp.broadcast_to(dinv, (RB, 8))


def _y_call(deg, x_p):
    return pl.pallas_call(
        _y_body,
        grid=(NPAD // RB,),
        in_specs=[
            pl.BlockSpec((NCORES, RB), lambda i: (0, i)),
            pl.BlockSpec((RB, D_IN), lambda i: (i, 0)),
        ],
        out_specs=[pl.BlockSpec((RB, D_IN), lambda i: (i, 0)),
                   pl.BlockSpec((RB, 8), lambda i: (i, 0))],
        out_shape=[jax.ShapeDtypeStruct((NPAD, D_IN), jnp.float32),
                   jax.ShapeDtypeStruct((NPAD, 8), jnp.float32)],
    )(deg, x_p)


RD = 1000
NBLK = N // RD


def _dense_body(parts_ref, y_ref, dinv_ref, batch_ref, w0_ref, sh0_ref,
                w1_ref, sh1_ref, w2_ref, b2_ref, out_ref,
                pooled_acc, cnt_acc):
    i = pl.program_id(0)

    @pl.when(i == 0)
    def _init():
        pooled_acc[...] = jnp.zeros_like(pooled_acc)
        cnt_acc[...] = jnp.zeros_like(cnt_acc)

    dinv = dinv_ref[...][:, 0:1]
    p = parts_ref[...]
    s = p[0] + p[1] + y_ref[...]
    agg = s * dinv
    h = jnp.dot(agg, w0_ref[...], preferred_element_type=jnp.float32)
    h = h * sh0_ref[0:1, :] + sh0_ref[1:2, :]
    h = jnp.maximum(h, 0.0)

    bids = batch_ref[...].reshape(1, RD)
    gid = lax.broadcasted_iota(jnp.int32, (NG, RD), 0)
    oh = (gid == bids).astype(jnp.float32)
    pooled_acc[...] += lax.dot_general(
        oh, h, (((1,), (0,)), ((), ())),
        preferred_element_type=jnp.float32)
    cnt_acc[...] += jnp.sum(oh, axis=1, keepdims=True)

    @pl.when(i == NBLK - 1)
    def _finish():
        pooled = pooled_acc[...] / jnp.maximum(cnt_acc[...], 1.0)
        z = jnp.dot(pooled, w1_ref[...], preferred_element_type=jnp.float32)
        z = z * sh1_ref[0:1, :] + sh1_ref[1:2, :]
        z = jnp.maximum(z, 0.0)
        z = jnp.dot(z, w2_ref[...], preferred_element_type=jnp.float32)
        out_ref[...] = z + b2_ref[...]


def _dense_call(parts, y_p, dinv8, batch3, W0, sh0, W1, sh1, W2, b2r):
    return pl.pallas_call(
        _dense_body,
        grid=(NBLK,),
        in_specs=[
            pl.BlockSpec((NCORES, RD, D_IN), lambda i: (0, i, 0)),
            pl.BlockSpec((RD, D_IN), lambda i: (i, 0)),
            pl.BlockSpec((RD, 8), lambda i: (i, 0)),
            pl.BlockSpec((1, 1, RD), lambda i: (i, 0, 0)),
            pl.BlockSpec((D_IN, D_H), lambda i: (0, 0)),
            pl.BlockSpec((2, D_H), lambda i: (0, 0)),
            pl.BlockSpec((D_H, D_C1), lambda i: (0, 0)),
            pl.BlockSpec((2, D_C1), lambda i: (0, 0)),
            pl.BlockSpec((D_C1, NCLS), lambda i: (0, 0)),
            pl.BlockSpec((1, NCLS), lambda i: (0, 0)),
        ],
        out_specs=pl.BlockSpec((NG, NCLS), lambda i: (0, 0)),
        out_shape=jax.ShapeDtypeStruct((NG, NCLS), jnp.float32),
        scratch_shapes=[
            pltpu.VMEM((NG, D_H), jnp.float32),
            pltpu.VMEM((NG, 1), jnp.float32),
        ],
    )(parts, y_p, dinv8, batch3, W0, sh0, W1, sh1, W2, b2r)


def kernel(x, edge_index, batch, W0, b0, bn0_g, bn0_b, W1, b1, bn1_g, bn1_b,
           W2, b2):
    f32 = jnp.float32
    pad = jnp.full((EPAD - E,), N, dtype=jnp.int32)
    src_p = jnp.concatenate([edge_index[0], pad])
    dst_p = jnp.concatenate([edge_index[1], pad])
    x_p = jnp.zeros((NPAD, D_IN), f32).at[:N].set(x)
    batch3 = batch.reshape(NBLK, 1, RD)

    ones = jnp.ones((CHUNK,), f32)
    zeros_deg = jnp.zeros((RPT,), f32)
    zeros_row = jnp.zeros((RPT, D_IN), f32)

    deg_call, agg_call = _sc_calls()
    deg = deg_call(dst_p, zeros_deg, ones)
    y_p, dinv8 = _y_call(deg, x_p)
    parts = agg_call(src_p, dst_p, y_p, zeros_row)

    c = lax.rsqrt(jnp.asarray(1.0 + 1e-5, f32))
    sh0 = jnp.stack([c * bn0_g, b0 * c * bn0_g + bn0_b])
    sh1 = jnp.stack([c * bn1_g, b1 * c * bn1_g + bn1_b])
    b2r = b2.reshape(1, NCLS)

    return _dense_call(parts, y_p, dinv8, batch3, W0, sh0, W1, sh1, W2, b2r)


# speedup vs baseline: 23.0806x; 1.0021x over previous
import functools

import jax
import jax.numpy as jnp
from jax import lax
from jax.experimental import pallas as pl
from jax.experimental.pallas import tpu as pltpu
from jax.experimental.pallas import tpu_sc as plsc

N = 10000
E = 320000
D_IN = 128
D_H = 512
D_C1 = 1024
NCLS = 128
NG = 64

NCORES = 2
NSUB = 16
NW = NCORES * NSUB

CHUNK = 64
NCH0 = 240
NCH1 = 80
EPT0 = NCH0 * CHUNK
EPT1 = NCH1 * CHUNK
EPAD = NSUB * (EPT0 + EPT1)

NPAD = 10240
RPT = NPAD // NSUB


def _tile_params(c, s):
    nout = jnp.where(c == 0, NCH0 // 8, NCH1 // 8)
    base = jnp.where(c == 0, s * EPT0, NSUB * EPT0 + s * EPT1)
    return nout, base


def _deg_body(dst_hbm, zeros_hbm, ones_hbm, out_hbm, ij, ones_v, deg_sh,
              sem_i, sem_s):
    c = lax.axis_index("c")
    s = lax.axis_index("s")
    nout, base = _tile_params(c, s)
    pltpu.sync_copy(ones_hbm, ones_v)
    pltpu.sync_copy(zeros_hbm, deg_sh.at[pl.ds(s * RPT, RPT)])

    def load(j, slot):
        pltpu.async_copy(dst_hbm.at[pl.ds(base + j * CHUNK, CHUNK)],
                         ij[slot], sem_i[slot])

    def wait_idx(slot):
        pltpu.make_async_copy(dst_hbm.at[pl.ds(0, CHUNK)],
                              ij[slot], sem_i[slot]).wait()

    for k in range(4):
        load(k, k)
    plsc.subcore_barrier()

    def outer(g, carry):
        for b in range(8):
            s8, s4 = b % 8, b % 4
            wait_idx(s8)

            def w_s():
                pltpu.make_async_copy(
                    ones_v, deg_sh.at[ij[(b + 4) % 8]],
                    sem_s[s4]).wait()

            if b >= 4:
                w_s()
            else:
                pl.when(g >= 1)(w_s)
            pltpu.async_copy(ones_v, deg_sh.at[ij[s8]], sem_s[s4],
                             add=True)

            def l_n():
                load(g * 8 + b + 4, (b + 4) % 8)

            if b < 4:
                l_n()
            else:
                pl.when(g < nout - 1)(l_n)
        return carry

    lax.fori_loop(0, nout, outer, 0)
    for k in range(4):
        pltpu.make_async_copy(ones_v, deg_sh.at[ij[(k + 4) % 8]],
                              sem_s[k]).wait()
    plsc.subcore_barrier()
    pltpu.sync_copy(deg_sh.at[pl.ds(s * RPT, RPT)],
                    out_hbm.at[c, pl.ds(s * RPT, RPT)])


def _agg_body(src_hbm, dst_hbm, y_hbm, zeros_hbm, out_hbm, isrc, idst, rows,
              s_sh, sem_i, sem_g, sem_s):
    c = lax.axis_index("c")
    s = lax.axis_index("s")
    nout, base = _tile_params(c, s)
    pltpu.sync_copy(zeros_hbm, s_sh.at[pl.ds(s * RPT, RPT)])

    def load(j, slot):
        off = base + j * CHUNK
        pltpu.async_copy(src_hbm.at[pl.ds(off, CHUNK)], isrc[slot],
                         sem_i[slot])
        pltpu.async_copy(dst_hbm.at[pl.ds(off, CHUNK)], idst[slot],
                         sem_i[slot])

    def wait_idx(slot):
        pltpu.make_async_copy(src_hbm.at[pl.ds(0, CHUNK)], isrc[slot],
                              sem_i[slot]).wait()
        pltpu.make_async_copy(dst_hbm.at[pl.ds(0, CHUNK)], idst[slot],
                              sem_i[slot]).wait()

    def gather(slot8, slot4):
        pltpu.async_copy(y_hbm.at[isrc[slot8]], rows[slot4], sem_g[slot4])

    for k in range(6):
        load(k, k)
    wait_idx(0)
    gather(0, 0)
    wait_idx(1)
    gather(1, 1)
    plsc.subcore_barrier()

    def outer(g, carry):
        for b in range(8):
            s8, s4 = b % 8, b % 4
            pltpu.make_async_copy(y_hbm.at[isrc[s8]], rows[s4],
                                  sem_g[s4]).wait()
            pltpu.async_copy(rows[s4], s_sh.at[idst[s8]], sem_s[s4],
                             add=True)

            def w_s():
                pltpu.make_async_copy(
                    rows[(b + 2) % 4], s_sh.at[idst[(b + 6) % 8]],
                    sem_s[(b + 2) % 4]).wait()

            if b >= 2:
                w_s()
            else:
                pl.when(g >= 1)(w_s)

            def g_n():
                wait_idx((b + 2) % 8)
                gather((b + 2) % 8, (b + 2) % 4)

            if b < 6:
                g_n()
            else:
                pl.when(g < nout - 1)(g_n)

            def l_n():
                load(g * 8 + b + 6, (b + 6) % 8)

            if b < 2:
                l_n()
            else:
                pl.when(g < nout - 1)(l_n)
        return carry

    lax.fori_loop(0, nout, outer, 0)
    pltpu.make_async_copy(rows[2], s_sh.at[idst[6]], sem_s[2]).wait()
    pltpu.make_async_copy(rows[3], s_sh.at[idst[7]], sem_s[3]).wait()
    plsc.subcore_barrier()
    pltpu.sync_copy(s_sh.at[pl.ds(s * RPT, RPT)],
                    out_hbm.at[c, pl.ds(s * RPT, RPT)])


@functools.cache
def _sc_calls():
    mesh = plsc.VectorSubcoreMesh(core_axis_name="c", subcore_axis_name="s",
                                  num_cores=NCORES, num_subcores=NSUB)
    deg_call = pl.kernel(
        _deg_body,
        out_type=jax.ShapeDtypeStruct((NCORES, NPAD), jnp.float32),
        mesh=mesh,
        scratch_types=[
            [pltpu.VMEM((CHUNK,), jnp.int32) for _ in range(8)],
            pltpu.VMEM((CHUNK,), jnp.float32),
            pltpu.VMEM_SHARED((NPAD,), jnp.float32),
            [pltpu.SemaphoreType.DMA for _ in range(8)],
            [pltpu.SemaphoreType.DMA for _ in range(4)],
        ],
    )
    agg_call = pl.kernel(
        _agg_body,
        out_type=jax.ShapeDtypeStruct((NCORES, NPAD, D_IN), jnp.float32),
        mesh=mesh,
        scratch_types=[
            [pltpu.VMEM((CHUNK,), jnp.int32) for _ in range(8)],
            [pltpu.VMEM((CHUNK,), jnp.int32) for _ in range(8)],
            [pltpu.VMEM((CHUNK, D_IN), jnp.float32) for _ in range(4)],
            pltpu.VMEM_SHARED((NPAD, D_IN), jnp.float32),
            [pltpu.SemaphoreType.DMA for _ in range(8)],
            [pltpu.SemaphoreType.DMA for _ in range(4)],
            [pltpu.SemaphoreType.DMA for _ in range(4)],
        ],
    )
    return deg_call, agg_call


RB = 1024


def _y_body(deg_ref, x_ref, y_ref, dinv_ref):
    d = deg_ref[...]
    cnt = d[0:1, :] + d[1:2, :] + 1.0
    dinv = jnp.transpose(lax.rsqrt(cnt))
    y_ref[...] = x_ref[...] * dinv
    dinv_ref[...] = jnp.broadcast_to(dinv, (RB, 8))


def _y_call(deg, x_p):
    return pl.pallas_call(
        _y_body,
        grid=(NPAD // RB,),
        in_specs=[
            pl.BlockSpec((NCORES, RB), lambda i: (0, i)),
            pl.BlockSpec((RB, D_IN), lambda i: (i, 0)),
        ],
        out_specs=[pl.BlockSpec((RB, D_IN), lambda i: (i, 0)),
                   pl.BlockSpec((RB, 8), lambda i: (i, 0))],
        out_shape=[jax.ShapeDtypeStruct((NPAD, D_IN), jnp.float32),
                   jax.ShapeDtypeStruct((NPAD, 8), jnp.float32)],
    )(deg, x_p)


RD = 1000
NBLK = N // RD


def _dense_body(parts_ref, y_ref, dinv_ref, batch_ref, w0_ref, sh0_ref,
                w1_ref, sh1_ref, w2_ref, b2_ref, out_ref,
                pooled_acc, cnt_acc):
    i = pl.program_id(0)

    @pl.when(i == 0)
    def _init():
        pooled_acc[...] = jnp.zeros_like(pooled_acc)
        cnt_acc[...] = jnp.zeros_like(cnt_acc)

    dinv = dinv_ref[...][:, 0:1]
    p = parts_ref[...]
    s = p[0] + p[1] + y_ref[...]
    agg = s * dinv
    h = jnp.dot(agg, w0_ref[...], preferred_element_type=jnp.float32)
    h = h * sh0_ref[0:1, :] + sh0_ref[1:2, :]
    h = jnp.maximum(h, 0.0)

    bids = batch_ref[...].reshape(1, RD)
    gid = lax.broadcasted_iota(jnp.int32, (NG, RD), 0)
    oh = (gid == bids).astype(jnp.float32)
    pooled_acc[...] += lax.dot_general(
        oh, h, (((1,), (0,)), ((), ())),
        preferred_element_type=jnp.float32)
    cnt_acc[...] += jnp.sum(oh, axis=1, keepdims=True)

    @pl.when(i == NBLK - 1)
    def _finish():
        pooled = pooled_acc[...] / jnp.maximum(cnt_acc[...], 1.0)
        z = jnp.dot(pooled, w1_ref[...], preferred_element_type=jnp.float32)
        z = z * sh1_ref[0:1, :] + sh1_ref[1:2, :]
        z = jnp.maximum(z, 0.0)
        z = jnp.dot(z, w2_ref[...], preferred_element_type=jnp.float32)
        out_ref[...] = z + b2_ref[...]


def _dense_call(parts, y_p, dinv8, batch3, W0, sh0, W1, sh1, W2, b2r):
    return pl.pallas_call(
        _dense_body,
        grid=(NBLK,),
        in_specs=[
            pl.BlockSpec((NCORES, RD, D_IN), lambda i: (0, i, 0)),
            pl.BlockSpec((RD, D_IN), lambda i: (i, 0)),
            pl.BlockSpec((RD, 8), lambda i: (i, 0)),
            pl.BlockSpec((1, 1, RD), lambda i: (i, 0, 0)),
            pl.BlockSpec((D_IN, D_H), lambda i: (0, 0)),
            pl.BlockSpec((2, D_H), lambda i: (0, 0)),
            pl.BlockSpec((D_H, D_C1), lambda i: (0, 0)),
            pl.BlockSpec((2, D_C1), lambda i: (0, 0)),
            pl.BlockSpec((D_C1, NCLS), lambda i: (0, 0)),
            pl.BlockSpec((1, NCLS), lambda i: (0, 0)),
        ],
        out_specs=pl.BlockSpec((NG, NCLS), lambda i: (0, 0)),
        out_shape=jax.ShapeDtypeStruct((NG, NCLS), jnp.float32),
        scratch_shapes=[
            pltpu.VMEM((NG, D_H), jnp.float32),
            pltpu.VMEM((NG, 1), jnp.float32),
        ],
    )(parts, y_p, dinv8, batch3, W0, sh0, W1, sh1, W2, b2r)


def kernel(x, edge_index, batch, W0, b0, bn0_g, bn0_b, W1, b1, bn1_g, bn1_b,
           W2, b2):
    f32 = jnp.float32
    pad = jnp.full((EPAD - E,), N, dtype=jnp.int32)
    src_p = jnp.concatenate([edge_index[0], pad])
    dst_p = jnp.concatenate([edge_index[1], pad])
    x_p = jnp.zeros((NPAD, D_IN), f32).at[:N].set(x)
    batch3 = batch.reshape(NBLK, 1, RD)

    ones = jnp.ones((CHUNK,), f32)
    zeros_deg = jnp.zeros((RPT,), f32)
    zeros_row = jnp.zeros((RPT, D_IN), f32)

    deg_call, agg_call = _sc_calls()
    deg = deg_call(dst_p, zeros_deg, ones)
    y_p, dinv8 = _y_call(deg, x_p)
    parts = agg_call(src_p, dst_p, y_p, zeros_row)

    c = lax.rsqrt(jnp.asarray(1.0 + 1e-5, f32))
    sh0 = jnp.stack([c * bn0_g, b0 * c * bn0_g + bn0_b])
    sh1 = jnp.stack([c * bn1_g, b1 * c * bn1_g + bn1_b])
    b2r = b2.reshape(1, NCLS)

    return _dense_call(parts, y_p, dinv8, batch3, W0, sh0, W1, sh1, W2, b2r)
